# baseline (speedup 1.0000x reference)
import functools

import jax
import jax.numpy as jnp
from jax import lax
from jax.experimental import pallas as pl
from jax.experimental.pallas import tpu as pltpu
from jax.experimental.pallas import tpu_sc as plsc

B, C_IN, T = 64, 128, 512
D = 256
K = 8192
COMMIT = 0.25

KC = 2048
NKC = K // KC
N = B * T


DA = D + 8


def _vq_body(x_ref, wenc_ref, benc_ref, cb_ref, idx_ref, md_ref, c2_ref):
    b = pl.program_id(0)

    @pl.when(b == 0)
    def _compute_c2():
        for kc in range(NKC):
            cb = cb_ref[pl.ds(kc * KC, KC), :]
            c2_ref[pl.ds(kc * KC, KC), :] = jnp.sum(cb * cb, axis=1,
                                                    keepdims=True)

    x = x_ref[0]
    zT = (lax.dot(wenc_ref[...], x, preferred_element_type=jnp.float32)
          + benc_ref[...])
    z2 = jnp.sum(zT * zT, axis=0, keepdims=True)
    zTm2 = -2.0 * zT

    minv = mini = None
    for kc in range(NKC):
        cb = cb_ref[pl.ds(kc * KC, KC), :]
        mm = lax.dot(cb, zTm2, preferred_element_type=jnp.float32)
        s = mm + c2_ref[pl.ds(kc * KC, KC), :]
        m = jnp.min(s, axis=0, keepdims=True)
        am = jnp.argmin(s, axis=0).astype(jnp.int32)[None, :] + kc * KC
        if minv is None:
            minv, mini = m, am
        else:
            upd = m < minv
            minv, mini = jnp.where(upd, m, minv), jnp.where(upd, am, mini)
    idx_ref[0] = mini
    md_ref[0] = minv + z2


def _vq_call(inp, W_enc, b_enc2, codebook, off, nb):
    return pl.pallas_call(
        _vq_body,
        grid=(nb,),
        in_specs=[
            pl.BlockSpec((1, C_IN, T), lambda b: (b + off, 0, 0)),
            pl.BlockSpec((D, C_IN), lambda b: (0, 0)),
            pl.BlockSpec((D, 1), lambda b: (0, 0)),
            pl.BlockSpec((K, D), lambda b: (0, 0)),
        ],
        out_specs=[
            pl.BlockSpec((1, 1, T), lambda b: (b, 0, 0)),
            pl.BlockSpec((1, 1, T), lambda b: (b, 0, 0)),
        ],
        out_shape=[
            jax.ShapeDtypeStruct((nb, 1, T), jnp.int32),
            jax.ShapeDtypeStruct((nb, 1, T), jnp.float32),
        ],
        scratch_shapes=[pltpu.VMEM((K, 1), jnp.float32)],
    )(inp, W_enc, b_enc2, codebook)


def _dec_body(cb_ref, wdec_ref, bdec_ref, out_ref):
    out_ref[...] = (lax.dot_general(
        cb_ref[...], wdec_ref[...], (((1,), (1,)), ((), ())),
        preferred_element_type=jnp.float32) + bdec_ref[...])


def _dec_call(codebook, W_dec, b_dec2):
    return pl.pallas_call(
        _dec_body,
        grid=(NKC,),
        in_specs=[
            pl.BlockSpec((KC, D), lambda i: (i, 0)),
            pl.BlockSpec((C_IN, D), lambda i: (0, 0)),
            pl.BlockSpec((1, C_IN), lambda i: (0, 0)),
        ],
        out_specs=pl.BlockSpec((KC, C_IN), lambda i: (i, 0)),
        out_shape=jax.ShapeDtypeStruct((K, C_IN), jnp.float32),
    )(codebook, W_dec, b_dec2)


_NW = 32
_CH = 128


def _gather_call(table, idx_flat):
    n = idx_flat.shape[0]
    bpw = n // _NW
    nch = bpw // _CH
    mesh = plsc.VectorSubcoreMesh(core_axis_name="c", subcore_axis_name="s")

    @functools.partial(
        pl.kernel, mesh=mesh,
        out_type=jax.ShapeDtypeStruct((n, C_IN), jnp.float32),
        scratch_types=[
            pltpu.VMEM((bpw,), jnp.int32),
            pltpu.VMEM((_CH, C_IN), jnp.float32),
            pltpu.VMEM((_CH, C_IN), jnp.float32),
            pltpu.SemaphoreType.DMA,
            pltpu.SemaphoreType.DMA,
        ],
    )
    def k(table_hbm, idx_hbm, out_hbm, idx_v, rows0, rows1, sem_g, sem_s):
        wid = lax.axis_index("s") * 2 + lax.axis_index("c")
        base = wid * bpw
        rows = [rows0, rows1]
        pltpu.sync_copy(idx_hbm.at[pl.ds(base, bpw)], idx_v)
        pltpu.async_copy(table_hbm.at[idx_v.at[pl.ds(0, _CH)]], rows[0], sem_g)
        for c in range(nch):
            pltpu.make_async_copy(table_hbm.at[idx_v.at[pl.ds(c * _CH, _CH)]],
                                  rows[c % 2], sem_g).wait()
            if c + 1 < nch:
                if c >= 1:
                    pltpu.make_async_copy(rows[(c + 1) % 2],
                                          out_hbm.at[pl.ds(base + (c - 1) * _CH, _CH)],
                                          sem_s).wait()
                pltpu.async_copy(
                    table_hbm.at[idx_v.at[pl.ds((c + 1) * _CH, _CH)]],
                    rows[(c + 1) % 2], sem_g)
            pltpu.async_copy(rows[c % 2],
                             out_hbm.at[pl.ds(base + c * _CH, _CH)], sem_s)
        pltpu.make_async_copy(rows[(nch - 2) % 2],
                              out_hbm.at[pl.ds(base + (nch - 2) * _CH, _CH)],
                              sem_s).wait()
        pltpu.make_async_copy(rows[(nch - 1) % 2],
                              out_hbm.at[pl.ds(base + (nch - 1) * _CH, _CH)],
                              sem_s).wait()

    return k(table, idx_flat)


def _norm_body(yf_ref, out_ref):
    y = yf_ref[0]
    mn = jnp.min(y, axis=0, keepdims=True)
    ysh = y - mn
    mx = jnp.max(ysh, axis=0, keepdims=True)
    out_ref[0] = jnp.transpose(ysh / (mx + 1e-08), (1, 0))


def _norm_call(y_flat3):
    nb = y_flat3.shape[0]
    return pl.pallas_call(
        _norm_body,
        grid=(nb,),
        in_specs=[pl.BlockSpec((1, T, C_IN), lambda b: (b, 0, 0))],
        out_specs=pl.BlockSpec((1, C_IN, T), lambda b: (b, 0, 0)),
        out_shape=jax.ShapeDtypeStruct((nb, C_IN, T), jnp.float32),
    )(y_flat3)


NSPLIT = 4


def kernel(input, W_enc, b_enc, codebook, W_dec, b_dec):
    cdec = _dec_call(codebook, W_dec, b_dec.reshape(1, C_IN))
    hb = B // NSPLIT
    idxs, mds = [], []
    for h in range(NSPLIT):
        idx3, md = _vq_call(input, W_enc, b_enc.reshape(D, 1), codebook,
                            h * hb, hb)
        idxs.append(idx3)
        mds.append(md)
    ys = []
    for h in range(NSPLIT):
        y_flat = _gather_call(cdec, idxs[h].reshape(hb * T))
        ys.append(_norm_call(y_flat.reshape(hb, T, C_IN)))
    y = jnp.concatenate(ys, axis=0)
    vq_loss = (COMMIT / (N * D)) * sum(jnp.sum(md) for md in mds)
    return (y, vq_loss)


# speedup vs baseline: 1.0279x; 1.0279x over previous
import functools

import jax
import jax.numpy as jnp
from jax import lax
from jax.experimental import pallas as pl
from jax.experimental.pallas import tpu as pltpu
from jax.experimental.pallas import tpu_sc as plsc

B, C_IN, T = 64, 128, 512
D = 256
K = 8192
COMMIT = 0.25

KC = 2048
NKC = K // KC
N = B * T


DA = D + 8


def _vq_body(x_ref, wenc_ref, benc_ref, cb_ref, idx_ref, md_ref, c2_ref):
    b = pl.program_id(0)

    @pl.when(b == 0)
    def _compute_c2():
        for kc in range(NKC):
            cb = cb_ref[pl.ds(kc * KC, KC), :]
            c2_ref[pl.ds(kc * KC, KC), :] = jnp.sum(cb * cb, axis=1,
                                                    keepdims=True)

    x = x_ref[0]
    zT = (lax.dot(wenc_ref[...], x, preferred_element_type=jnp.float32)
          + benc_ref[...])
    z2 = jnp.sum(zT * zT, axis=0, keepdims=True)
    zTm2 = -2.0 * zT

    minv = mini = None
    for kc in range(NKC):
        cb = cb_ref[pl.ds(kc * KC, KC), :]
        mm = lax.dot(cb, zTm2, preferred_element_type=jnp.float32)
        s = mm + c2_ref[pl.ds(kc * KC, KC), :]
        m = jnp.min(s, axis=0, keepdims=True)
        am = jnp.argmin(s, axis=0).astype(jnp.int32)[None, :] + kc * KC
        if minv is None:
            minv, mini = m, am
        else:
            upd = m < minv
            minv, mini = jnp.where(upd, m, minv), jnp.where(upd, am, mini)
    idx_ref[0] = mini
    md_ref[0] = minv + z2


def _vq_call(inp, W_enc, b_enc2, codebook, off, nb):
    return pl.pallas_call(
        _vq_body,
        grid=(nb,),
        in_specs=[
            pl.BlockSpec((1, C_IN, T), lambda b: (b + off, 0, 0)),
            pl.BlockSpec((D, C_IN), lambda b: (0, 0)),
            pl.BlockSpec((D, 1), lambda b: (0, 0)),
            pl.BlockSpec((K, D), lambda b: (0, 0)),
        ],
        out_specs=[
            pl.BlockSpec((1, 1, T), lambda b: (b, 0, 0)),
            pl.BlockSpec((1, 1, T), lambda b: (b, 0, 0)),
        ],
        out_shape=[
            jax.ShapeDtypeStruct((nb, 1, T), jnp.int32),
            jax.ShapeDtypeStruct((nb, 1, T), jnp.float32),
        ],
        scratch_shapes=[pltpu.VMEM((K, 1), jnp.float32)],
    )(inp, W_enc, b_enc2, codebook)


def _dec_body(cb_ref, wdec_ref, bdec_ref, out_ref):
    out_ref[...] = (lax.dot_general(
        cb_ref[...], wdec_ref[...], (((1,), (1,)), ((), ())),
        preferred_element_type=jnp.float32) + bdec_ref[...])


def _dec_call(codebook, W_dec, b_dec2):
    return pl.pallas_call(
        _dec_body,
        grid=(NKC,),
        in_specs=[
            pl.BlockSpec((KC, D), lambda i: (i, 0)),
            pl.BlockSpec((C_IN, D), lambda i: (0, 0)),
            pl.BlockSpec((1, C_IN), lambda i: (0, 0)),
        ],
        out_specs=pl.BlockSpec((KC, C_IN), lambda i: (i, 0)),
        out_shape=jax.ShapeDtypeStruct((K, C_IN), jnp.float32),
    )(codebook, W_dec, b_dec2)


_NW = 32
_CH = 256


def _gather_call(table, idx_flat):
    n = idx_flat.shape[0]
    bpw = n // _NW
    nch = bpw // _CH
    mesh = plsc.VectorSubcoreMesh(core_axis_name="c", subcore_axis_name="s")

    @functools.partial(
        pl.kernel, mesh=mesh,
        out_type=jax.ShapeDtypeStruct((n, C_IN), jnp.float32),
        scratch_types=[
            pltpu.VMEM((bpw,), jnp.int32),
            pltpu.VMEM((_CH, C_IN), jnp.float32),
            pltpu.VMEM((_CH, C_IN), jnp.float32),
            pltpu.SemaphoreType.DMA,
            pltpu.SemaphoreType.DMA,
        ],
    )
    def k(table_hbm, idx_hbm, out_hbm, idx_v, rows0, rows1, sem_g, sem_s):
        wid = lax.axis_index("s") * 2 + lax.axis_index("c")
        base = wid * bpw
        rows = [rows0, rows1]
        pltpu.sync_copy(idx_hbm.at[pl.ds(base, bpw)], idx_v)
        pltpu.async_copy(table_hbm.at[idx_v.at[pl.ds(0, _CH)]], rows[0], sem_g)
        for c in range(nch):
            pltpu.make_async_copy(table_hbm.at[idx_v.at[pl.ds(c * _CH, _CH)]],
                                  rows[c % 2], sem_g).wait()
            if c + 1 < nch:
                if c >= 1:
                    pltpu.make_async_copy(rows[(c + 1) % 2],
                                          out_hbm.at[pl.ds(base + (c - 1) * _CH, _CH)],
                                          sem_s).wait()
                pltpu.async_copy(
                    table_hbm.at[idx_v.at[pl.ds((c + 1) * _CH, _CH)]],
                    rows[(c + 1) % 2], sem_g)
            pltpu.async_copy(rows[c % 2],
                             out_hbm.at[pl.ds(base + c * _CH, _CH)], sem_s)
        pltpu.make_async_copy(rows[(nch - 2) % 2],
                              out_hbm.at[pl.ds(base + (nch - 2) * _CH, _CH)],
                              sem_s).wait()
        pltpu.make_async_copy(rows[(nch - 1) % 2],
                              out_hbm.at[pl.ds(base + (nch - 1) * _CH, _CH)],
                              sem_s).wait()

    return k(table, idx_flat)


def _norm_body(yf_ref, out_ref):
    y = yf_ref[0]
    mn = jnp.min(y, axis=0, keepdims=True)
    ysh = y - mn
    mx = jnp.max(ysh, axis=0, keepdims=True)
    out_ref[0] = jnp.transpose(ysh / (mx + 1e-08), (1, 0))


def _norm_call(y_flat3):
    nb = y_flat3.shape[0]
    return pl.pallas_call(
        _norm_body,
        grid=(nb,),
        in_specs=[pl.BlockSpec((1, T, C_IN), lambda b: (b, 0, 0))],
        out_specs=pl.BlockSpec((1, C_IN, T), lambda b: (b, 0, 0)),
        out_shape=jax.ShapeDtypeStruct((nb, C_IN, T), jnp.float32),
    )(y_flat3)


NSPLIT = 2


def kernel(input, W_enc, b_enc, codebook, W_dec, b_dec):
    cdec = _dec_call(codebook, W_dec, b_dec.reshape(1, C_IN))
    hb = B // NSPLIT
    idxs, mds = [], []
    for h in range(NSPLIT):
        idx3, md = _vq_call(input, W_enc, b_enc.reshape(D, 1), codebook,
                            h * hb, hb)
        idxs.append(idx3)
        mds.append(md)
    ys = []
    for h in range(NSPLIT):
        y_flat = _gather_call(cdec, idxs[h].reshape(hb * T))
        ys.append(_norm_call(y_flat.reshape(hb, T, C_IN)))
    y = jnp.concatenate(ys, axis=0)
    vq_loss = (COMMIT / (N * D)) * sum(jnp.sum(md) for md in mds)
    return (y, vq_loss)
